# Initial kernel scaffold; baseline (speedup 1.0000x reference)
import jax
import jax.numpy as jnp
from jax.experimental import pallas as pl


def kernel(x, batch, W, b):
    raise NotImplementedError("write your pallas kernel here")


# speedup vs baseline: 8.4069x; 8.4069x over previous
import jax
import jax.numpy as jnp
from jax.experimental import pallas as pl

_S = 512
_C = 6
_P = 8


def _body(ids_ref, x_ref, W_ref, b_ref, out_ref):
    i = pl.program_id(0)
    nb = pl.num_programs(0)

    @pl.when(i == 0)
    def _init():
        out_ref[...] = jnp.zeros_like(out_ref)

    x = x_ref[...]
    Wp = W_ref[...]
    y = jax.lax.dot_general(x, Wp, (((1,), (1,)), ((), ())),
                            preferred_element_type=jnp.float32)
    r = y.shape[0]
    col = jax.lax.broadcasted_iota(jnp.int32, (r, _P), 1)
    y = y + (col == _C).astype(jnp.float32)

    ids = ids_ref[0, 0, :]
    seg = jax.lax.broadcasted_iota(jnp.int32, (_S, r), 0)
    oh = (ids[None, :] == seg).astype(jnp.float32)
    out_ref[...] += jax.lax.dot_general(oh, y, (((1,), (0,)), ((), ())),
                                        preferred_element_type=jnp.float32)

    @pl.when(i == nb - 1)
    def _fin():
        acc = out_ref[...]
        cnt = jnp.clip(acc[:, _C:_C + 1], 1.0, None)
        z = acc / cnt + b_ref[...]
        out_ref[...] = jax.nn.sigmoid(z)


def kernel(x, batch, W, b):
    n, d = x.shape
    r = 0
    for cand in range(128, 4097, 128):
        if n % cand == 0:
            r = cand
    if r == 0:
        for cand in range(8, 4097, 8):
            if n % cand == 0:
                r = cand
    nb = n // r

    ids = batch.astype(jnp.int32).reshape(nb, 1, r)
    Wp = jnp.zeros((_P, d), W.dtype).at[:_C].set(W)
    bp = jnp.zeros((1, _P), b.dtype).at[0, :_C].set(b)

    out = pl.pallas_call(
        _body,
        grid=(nb,),
        in_specs=[
            pl.BlockSpec((1, 1, r), lambda i: (i, 0, 0)),
            pl.BlockSpec((r, d), lambda i: (i, 0)),
            pl.BlockSpec((_P, d), lambda i: (0, 0)),
            pl.BlockSpec((1, _P), lambda i: (0, 0)),
        ],
        out_specs=pl.BlockSpec((_S, _P), lambda i: (0, 0)),
        out_shape=jax.ShapeDtypeStruct((_S, _P), jnp.float32),
    )(ids, x, Wp, bp)
    return out[:, :_C]
